# Initial kernel scaffold; baseline (speedup 1.0000x reference)
import jax
import jax.numpy as jnp
from jax.experimental import pallas as pl


def kernel(x, table):
    raise NotImplementedError("write your pallas kernel here")


# speedup vs baseline: 1.6666x; 1.6666x over previous
import functools
import math

import jax
import jax.numpy as jnp
from jax import lax
from jax.experimental import pallas as pl
from jax.experimental.pallas import tpu as pltpu
from jax.experimental.pallas import tpu_sc as plsc

D_MODEL = 768
SCALE = math.sqrt(float(D_MODEL))

_NBUF = 2
_CHUNK = 32


@functools.cache
def _build(B: int, V: int, D: int):
    info = plsc.get_sparse_core_info()
    NC, NS, L = info.num_cores, info.num_subcores, info.num_lanes
    NW = NC * NS
    assert B % NW == 0
    b_per_w = B // NW
    C = _CHUNK
    assert b_per_w % C == 0 and D % L == 0
    n_chunks = b_per_w // C
    n_slices = D // L

    mesh = plsc.VectorSubcoreMesh(core_axis_name="c", subcore_axis_name="s")

    @functools.partial(
        pl.kernel,
        mesh=mesh,
        out_type=jax.ShapeDtypeStruct((B, D), jnp.float32),
        scratch_types=[
            pltpu.VMEM((b_per_w,), jnp.int32),
            [pltpu.VMEM((C, D), jnp.float32) for _ in range(_NBUF)],
            [pltpu.VMEM((C, D), jnp.float32) for _ in range(_NBUF)],
            [pltpu.SemaphoreType.DMA for _ in range(_NBUF)],
            [pltpu.SemaphoreType.DMA for _ in range(_NBUF)],
        ],
    )
    def emb_kernel(x_hbm, table_hbm, out_hbm, idx_v, ibufs, obufs, gsems, ssems):
        wid = lax.axis_index("s") * NC + lax.axis_index("c")
        base = wid * b_per_w

        pltpu.sync_copy(x_hbm.at[pl.ds(base, b_per_w)], idx_v)

        def start_gather(g, b):
            pltpu.make_async_copy(
                table_hbm.at[idx_v.at[pl.ds(g * C, C)]], ibufs[b], gsems[b]
            ).start()

        def wait_gather(g, b):
            pltpu.make_async_copy(
                table_hbm.at[idx_v.at[pl.ds(g * C, C)]], ibufs[b], gsems[b]
            ).wait()

        def start_scatter(g, b):
            pltpu.make_async_copy(
                obufs[b], out_hbm.at[pl.ds(base + g * C, C)], ssems[b]
            ).start()

        def wait_scatter(g, b):
            pltpu.make_async_copy(
                obufs[b], out_hbm.at[pl.ds(base + g * C, C)], ssems[b]
            ).wait()

        def scale_chunk(b):
            def row(r, _):
                for j in range(n_slices):
                    sl = (r, pl.ds(j * L, L))
                    obufs[b][sl] = ibufs[b][sl] * SCALE
                return _

            lax.fori_loop(0, C, row, 0, unroll=False)

        for b in range(_NBUF):
            start_gather(b, b)

        for g in range(_NBUF):
            b = g % _NBUF
            wait_gather(g, b)
            scale_chunk(b)
            start_scatter(g, b)
            start_gather(g + _NBUF, b)

        def outer(k, _):
            for b in range(_NBUF):
                g = _NBUF + k * _NBUF + b
                wait_gather(g, b)
                wait_scatter(g - _NBUF, b)
                scale_chunk(b)
                start_scatter(g, b)
                start_gather(g + _NBUF, b)
            return _

        n_steady = (n_chunks - 2 * _NBUF) // _NBUF
        lax.fori_loop(0, n_steady, outer, 0, unroll=False)

        for g in range(n_chunks - _NBUF, n_chunks):
            b = g % _NBUF
            wait_gather(g, b)
            wait_scatter(g - _NBUF, b)
            scale_chunk(b)
            start_scatter(g, b)

        for g in range(n_chunks - _NBUF, n_chunks):
            wait_scatter(g, g % _NBUF)

    return emb_kernel


def kernel(x, table):
    B = x.shape[0] * x.shape[1]
    V, D = table.shape
    idx = x.reshape(-1).astype(jnp.int32)
    out = _build(B, V, D)(idx, table)
    return out.reshape(x.shape[0], x.shape[1], D)
